# baseline (speedup 1.0000x reference)
import functools
import jax
import jax.numpy as jnp
from jax import lax
from jax.experimental import pallas as pl
from jax.experimental.pallas import tpu as pltpu
from jax.experimental.pallas import tpu_sc as plsc

_NTOT = 4 * 2048 * 4096
_NW = 32
_PER_W = _NTOT // _NW
_CH = 16384
_NPAIR = _PER_W // (2 * _CH)

_mesh = plsc.VectorSubcoreMesh(core_axis_name="c", subcore_axis_name="s")


def _accum_buf(buf, acc_s, acc_q):
    def body(k, carry):
        cs, cq = carry
        base = k * 128
        for j in range(8):
            v = buf[pl.ds(base + j * 16, 16)]
            cs = cs + v
            cq = cq + v * v
        return cs, cq
    return lax.fori_loop(0, _CH // 128, body, (acc_s, acc_q))


@functools.partial(
    pl.kernel,
    mesh=_mesh,
    out_type=[
        jax.ShapeDtypeStruct((_NW, 16), jnp.float32),
        jax.ShapeDtypeStruct((_NW, 16), jnp.float32),
    ],
    scratch_types=[
        pltpu.VMEM((_CH,), jnp.float32),
        pltpu.VMEM((_CH,), jnp.float32),
        pltpu.VMEM((16,), jnp.float32),
        pltpu.VMEM((16,), jnp.float32),
        pltpu.SemaphoreType.DMA,
        pltpu.SemaphoreType.DMA,
    ],
)
def _sc_stats(x_hbm, s_out, q_out, buf0, buf1, st_s, st_q, sem0, sem1):
    wid = lax.axis_index("s") * 2 + lax.axis_index("c")
    base = wid * _PER_W

    pltpu.async_copy(x_hbm.at[pl.ds(base, _CH)], buf0, sem0)

    def pair(j, carry):
        acc_s, acc_q = carry
        c0 = 2 * j
        pltpu.async_copy(x_hbm.at[pl.ds(base + (c0 + 1) * _CH, _CH)], buf1, sem1)
        pltpu.make_async_copy(x_hbm.at[pl.ds(0, _CH)], buf0, sem0).wait()
        acc_s, acc_q = _accum_buf(buf0, acc_s, acc_q)

        @pl.when(j < _NPAIR - 1)
        def _():
            pltpu.async_copy(x_hbm.at[pl.ds(base + (c0 + 2) * _CH, _CH)], buf0, sem0)

        pltpu.make_async_copy(x_hbm.at[pl.ds(0, _CH)], buf1, sem1).wait()
        acc_s, acc_q = _accum_buf(buf1, acc_s, acc_q)
        return acc_s, acc_q

    z = jnp.zeros((16,), jnp.float32)
    acc_s, acc_q = lax.fori_loop(0, _NPAIR, pair, (z, z))
    st_s[...] = acc_s
    st_q[...] = acc_q
    pltpu.sync_copy(st_s, s_out.at[wid])
    pltpu.sync_copy(st_q, q_out.at[wid])


@functools.partial(
    pl.kernel,
    mesh=_mesh,
    out_type=jax.ShapeDtypeStruct((_NTOT,), jnp.float32),
    scratch_types=[
        pltpu.VMEM((_CH,), jnp.float32),
        pltpu.VMEM((_CH,), jnp.float32),
        pltpu.VMEM((16,), jnp.float32),
        pltpu.VMEM((16,), jnp.float32),
        pltpu.SemaphoreType.DMA,
        pltpu.SemaphoreType.DMA,
        pltpu.SemaphoreType.DMA,
        pltpu.SemaphoreType.DMA,
        pltpu.SemaphoreType.DMA,
    ],
)
def _sc_apply(x_hbm, ab_hbm, o_hbm, buf0, buf1, va_ref, vb_ref,
              sem0, sem1, semo0, semo1, semab):
    wid = lax.axis_index("s") * 2 + lax.axis_index("c")
    base = wid * _PER_W

    pltpu.async_copy(ab_hbm.at[pl.ds(0, 16)], va_ref, semab)
    pltpu.async_copy(ab_hbm.at[pl.ds(16, 16)], vb_ref, semab)
    pltpu.async_copy(x_hbm.at[pl.ds(base, _CH)], buf0, sem0)
    pltpu.make_async_copy(ab_hbm.at[pl.ds(0, 16)], va_ref, semab).wait()
    pltpu.make_async_copy(ab_hbm.at[pl.ds(0, 16)], vb_ref, semab).wait()
    va = va_ref[...]
    vb = vb_ref[...]

    def transform(buf):
        def body(k, _):
            b = k * 128
            for j in range(8):
                v = buf[pl.ds(b + j * 16, 16)]
                buf[pl.ds(b + j * 16, 16)] = v * va + vb
            return 0
        lax.fori_loop(0, _CH // 128, body, 0)

    def pair(j, _):
        c0 = 2 * j
        pltpu.async_copy(x_hbm.at[pl.ds(base + (c0 + 1) * _CH, _CH)], buf1, sem1)
        pltpu.make_async_copy(x_hbm.at[pl.ds(0, _CH)], buf0, sem0).wait()

        @pl.when(j > 0)
        def _():
            pltpu.make_async_copy(buf0, o_hbm.at[pl.ds(0, _CH)], semo0).wait()

        transform(buf0)
        pltpu.async_copy(buf0, o_hbm.at[pl.ds(base + c0 * _CH, _CH)], semo0)

        pltpu.make_async_copy(x_hbm.at[pl.ds(0, _CH)], buf1, sem1).wait()

        @pl.when(j > 0)
        def _():
            pltpu.make_async_copy(buf1, o_hbm.at[pl.ds(0, _CH)], semo1).wait()

        transform(buf1)
        pltpu.async_copy(buf1, o_hbm.at[pl.ds(base + (c0 + 1) * _CH, _CH)], semo1)

        @pl.when(j < _NPAIR - 1)
        def _():
            pltpu.async_copy(x_hbm.at[pl.ds(base + (c0 + 2) * _CH, _CH)], buf0, sem0)

        return 0

    lax.fori_loop(0, _NPAIR, pair, 0)
    pltpu.make_async_copy(buf0, o_hbm.at[pl.ds(0, _CH)], semo0).wait()
    pltpu.make_async_copy(buf1, o_hbm.at[pl.ds(0, _CH)], semo1).wait()


def kernel(data, params, scalei, scaleo):
    x = data.reshape(_NTOT)
    s_p, q_p = _sc_stats(x)
    s = jnp.sum(s_p)
    q = jnp.sum(q_p)
    mean = s / _NTOT
    var = (q - s * s / _NTOT) / (_NTOT - 1)
    std = jnp.sqrt(var)
    t0 = params[0, 0, 0]
    v0 = params[1, 0, 0]
    ci = scalei.reshape(())
    co = scaleo.reshape(())
    amp = jnp.exp(v0 * jnp.sin(t0)) * ci
    alpha = jnp.sign(amp) * co / std
    beta = -mean * alpha
    ab = jnp.concatenate([jnp.full((16,), alpha), jnp.full((16,), beta)])
    out = _sc_apply(x, ab)
    return out.reshape(data.shape)


# speedup vs baseline: 3.1258x; 3.1258x over previous
import jax
import jax.numpy as jnp
from jax.experimental import pallas as pl
from jax.experimental.pallas import tpu as pltpu

_R = 8192
_C = 4096
_BR = 256
_NB = _R // _BR
_N = _R * _C


def _fused_kernel(sc_ref, x_ref, o_ref, acc_ref):
    p = pl.program_id(0)
    i = pl.program_id(1)

    @pl.when(p == 0)
    def _reduce():
        x = x_ref[...]
        s = jnp.sum(x)
        q = jnp.sum(x * x)

        @pl.when(i == 0)
        def _():
            acc_ref[0, 0] = 0.0
            acc_ref[0, 1] = 0.0

        acc_ref[0, 0] += s
        acc_ref[0, 1] += q

    @pl.when(p == 1)
    def _apply():
        @pl.when(i == 0)
        def _():
            s = acc_ref[0, 0]
            q = acc_ref[0, 1]
            mean = s / _N
            var = (q - s * s / _N) / (_N - 1)
            std = jnp.sqrt(var)
            t0 = sc_ref[0]
            v0 = sc_ref[1]
            ci = sc_ref[2]
            co = sc_ref[3]
            amp = jnp.exp(v0 * jnp.sin(t0)) * ci
            alpha = jnp.sign(amp) * co / std
            acc_ref[0, 2] = alpha
            acc_ref[0, 3] = -mean * alpha

        alpha = acc_ref[0, 2]
        beta = acc_ref[0, 3]
        o_ref[...] = x_ref[...] * alpha + beta


def kernel(data, params, scalei, scaleo):
    x = data.reshape(_R, _C)
    scalars = jnp.stack([
        params[0, 0, 0],
        params[1, 0, 0],
        scalei.reshape(()),
        scaleo.reshape(()),
    ])
    out = pl.pallas_call(
        _fused_kernel,
        grid=(2, _NB),
        in_specs=[
            pl.BlockSpec(memory_space=pltpu.SMEM),
            pl.BlockSpec((_BR, _C), lambda p, i: (i, 0)),
        ],
        out_specs=pl.BlockSpec((_BR, _C), lambda p, i: (i * p, 0)),
        out_shape=jax.ShapeDtypeStruct((_R, _C), jnp.float32),
        scratch_shapes=[pltpu.SMEM((1, 4), jnp.float32)],
    )(scalars, x)
    return out.reshape(data.shape)
